# baseline (speedup 1.0000x reference)
import functools

import jax
import jax.numpy as jnp
from jax import lax
from jax.experimental import pallas as pl
from jax.experimental.pallas import tpu as pltpu
from jax.experimental.pallas import tpu_sc as plsc

_N_BUF = 2


@functools.lru_cache(maxsize=None)
def _make_gather(B, V, D):
    info = plsc.get_sparse_core_info()
    NC, NS = info.num_cores, info.num_subcores
    NW = NC * NS
    assert B % NW == 0
    b_per_w = B // NW
    CHUNK = 800
    while b_per_w % CHUNK:
        CHUNK //= 2
    n_chunks = b_per_w // CHUNK

    mesh = plsc.VectorSubcoreMesh(core_axis_name="c", subcore_axis_name="s")

    @functools.partial(
        pl.kernel,
        mesh=mesh,
        out_type=jax.ShapeDtypeStruct((B, D), jnp.float32),
        compiler_params=pltpu.CompilerParams(use_tc_tiling_on_sc=False),
        scratch_types=[
            pltpu.VMEM((b_per_w,), jnp.int32),
            pltpu.VMEM((_N_BUF, CHUNK, D), jnp.float32),
            [pltpu.SemaphoreType.DMA] * _N_BUF,
            [pltpu.SemaphoreType.DMA] * _N_BUF,
        ],
    )
    def gather_kernel(idx_hbm, table_hbm, out_hbm, idx_v, rows_v, gsems, osems):
        wid = lax.axis_index("s") * NC + lax.axis_index("c")
        base = wid * b_per_w
        pltpu.sync_copy(idx_hbm.at[pl.ds(base, b_per_w)], idx_v)

        def start_gather(j):
            b = j % _N_BUF
            return pltpu.async_copy(
                table_hbm.at[idx_v.at[pl.ds(j * CHUNK, CHUNK)]],
                rows_v.at[b],
                gsems[b],
            )

        gather = start_gather(0)
        writes = [None] * n_chunks
        for j in range(n_chunks):
            b = j % _N_BUF
            gather.wait()
            if j + 1 < n_chunks:
                if j + 1 >= _N_BUF:
                    writes[j + 1 - _N_BUF].wait()
                gather = start_gather(j + 1)
            writes[j] = pltpu.async_copy(
                rows_v.at[b],
                out_hbm.at[pl.ds(base + j * CHUNK, CHUNK)],
                osems[b],
            )
        for j in range(max(0, n_chunks - _N_BUF), n_chunks):
            writes[j].wait()

    return gather_kernel


def kernel(X, table):
    batch, hist = X.shape
    V, D = table.shape
    B = batch * hist
    idx = lax.max(X.reshape(B).astype(jnp.int32), 0)
    half = B // 2
    fn = _make_gather(half, V, D)
    pieces = []
    for k in range(2):
        out_k = fn(lax.dynamic_slice_in_dim(idx, k * half, half), table)
        pieces.append(out_k.reshape(batch // 2, hist, D))
    return jnp.concatenate(pieces, axis=0)


# speedup vs baseline: 1.0665x; 1.0665x over previous
import functools

import jax
import jax.numpy as jnp
from jax import lax
from jax.experimental import pallas as pl
from jax.experimental.pallas import tpu as pltpu
from jax.experimental.pallas import tpu_sc as plsc

_N_BUF = 2


@functools.lru_cache(maxsize=None)
def _make_gather(B, V, D):
    info = plsc.get_sparse_core_info()
    NC, NS = info.num_cores, info.num_subcores
    NW = NC * NS
    assert B % NW == 0
    b_per_w = B // NW
    CHUNK = 800
    while b_per_w % CHUNK:
        CHUNK //= 2
    n_chunks = b_per_w // CHUNK

    mesh = plsc.VectorSubcoreMesh(core_axis_name="c", subcore_axis_name="s")

    @functools.partial(
        pl.kernel,
        mesh=mesh,
        out_type=jax.ShapeDtypeStruct((B, D), jnp.float32),
        compiler_params=pltpu.CompilerParams(use_tc_tiling_on_sc=False),
        scratch_types=[
            pltpu.VMEM((b_per_w,), jnp.int32),
            pltpu.VMEM((_N_BUF, CHUNK, D), jnp.float32),
            [pltpu.SemaphoreType.DMA] * _N_BUF,
            [pltpu.SemaphoreType.DMA] * _N_BUF,
        ],
    )
    def gather_kernel(idx_hbm, table_hbm, out_hbm, idx_v, rows_v, gsems, osems):
        wid = lax.axis_index("s") * NC + lax.axis_index("c")
        base = wid * b_per_w
        pltpu.sync_copy(idx_hbm.at[pl.ds(base, b_per_w)], idx_v)

        def start_gather(j):
            b = j % _N_BUF
            return pltpu.async_copy(
                table_hbm.at[idx_v.at[pl.ds(j * CHUNK, CHUNK)]],
                rows_v.at[b],
                gsems[b],
            )

        gather = start_gather(0)
        writes = [None] * n_chunks
        for j in range(n_chunks):
            b = j % _N_BUF
            gather.wait()
            if j + 1 < n_chunks:
                if j + 1 >= _N_BUF:
                    writes[j + 1 - _N_BUF].wait()
                gather = start_gather(j + 1)
            writes[j] = pltpu.async_copy(
                rows_v.at[b],
                out_hbm.at[pl.ds(base + j * CHUNK, CHUNK)],
                osems[b],
            )
        for j in range(max(0, n_chunks - _N_BUF), n_chunks):
            writes[j].wait()

    return gather_kernel


def kernel(X, table):
    batch, hist = X.shape
    V, D = table.shape
    B = batch * hist
    idx = lax.max(X.reshape(B).astype(jnp.int32), 0)
    out = _make_gather(B, V, D)(idx, table)
    return out.reshape(batch, hist, D)


# speedup vs baseline: 1.0693x; 1.0026x over previous
import functools

import jax
import jax.numpy as jnp
from jax import lax
from jax.experimental import pallas as pl
from jax.experimental.pallas import tpu as pltpu
from jax.experimental.pallas import tpu_sc as plsc

_N_BUF = 2


@functools.lru_cache(maxsize=None)
def _make_gather(B, V, D):
    info = plsc.get_sparse_core_info()
    NC, NS = info.num_cores, info.num_subcores
    NW = NC * NS
    assert B % NW == 0
    b_per_w = B // NW
    CHUNK = 800
    while b_per_w % CHUNK:
        CHUNK //= 2
    n_chunks = b_per_w // CHUNK

    mesh = plsc.VectorSubcoreMesh(core_axis_name="c", subcore_axis_name="s")

    @functools.partial(
        pl.kernel,
        mesh=mesh,
        out_type=jax.ShapeDtypeStruct((B, D), jnp.float32),
        compiler_params=pltpu.CompilerParams(
            use_tc_tiling_on_sc=False, needs_layout_passes=True
        ),
        scratch_types=[
            pltpu.VMEM((b_per_w,), jnp.int32),
            pltpu.VMEM((_N_BUF, CHUNK, D), jnp.float32),
            [pltpu.SemaphoreType.DMA] * _N_BUF,
            [pltpu.SemaphoreType.DMA] * _N_BUF,
        ],
    )
    def gather_kernel(idx_hbm, table_hbm, out_hbm, idx_v, rows_v, gsems, osems):
        wid = lax.axis_index("s") * NC + lax.axis_index("c")
        base = wid * b_per_w
        pltpu.sync_copy(idx_hbm.at[pl.ds(base, b_per_w)], idx_v)

        def start_gather(j):
            b = j % _N_BUF
            return pltpu.async_copy(
                table_hbm.at[idx_v.at[pl.ds(j * CHUNK, CHUNK)]],
                rows_v.at[b],
                gsems[b],
            )

        gather = start_gather(0)
        writes = [None] * n_chunks
        for j in range(n_chunks):
            b = j % _N_BUF
            gather.wait()
            if j + 1 < n_chunks:
                if j + 1 >= _N_BUF:
                    writes[j + 1 - _N_BUF].wait()
                gather = start_gather(j + 1)
            writes[j] = pltpu.async_copy(
                rows_v.at[b],
                out_hbm.at[pl.ds(base + j * CHUNK, CHUNK)],
                osems[b],
            )
        for j in range(max(0, n_chunks - _N_BUF), n_chunks):
            writes[j].wait()

    return gather_kernel


def kernel(X, table):
    batch, hist = X.shape
    V, D = table.shape
    B = batch * hist
    idx = lax.max(X.reshape(B).astype(jnp.int32), 0)
    out = _make_gather(B, V, D)(idx, table)
    return out.reshape(batch, hist, D)
